# baseline (speedup 1.0000x reference)
import functools

import jax
import jax.numpy as jnp
from jax import lax
from jax.experimental import pallas as pl
from jax.experimental.pallas import tpu as pltpu
from jax.experimental.pallas import tpu_sc as plsc
from jax._src.pallas import mpmd as plmpmd

SEQ = 4096
D = 1024
NC = 2
NS = 16
ROWS_PER_C = SEQ // NC

TEC_ROWS_PER_C = 1152
TEC_ROWS_PER_W = TEC_ROWS_PER_C // NS
TEC_CHUNKS = (8, 32, 32)
TEC_CHMAX = max(TEC_CHUNKS)
TEC_NCHUNK = len(TEC_CHUNKS)
TEC_OFFS = [sum(TEC_CHUNKS[:i]) for i in range(TEC_NCHUNK)]
TEC_NBUF = 3

SCS_ROWS_PER_C = ROWS_PER_C - TEC_ROWS_PER_C
SCS_CH = 128
SCS_NCHUNK = SCS_ROWS_PER_C // SCS_CH
SCS_NBUF = 4

_vmesh = plsc.VectorSubcoreMesh(core_axis_name="c", subcore_axis_name="s")
_smesh = plsc.ScalarSubcoreMesh(axis_name="c", num_cores=NC)

_N_TEC_SCRATCH = TEC_NBUF + 2 * TEC_NBUF


def _tec_fn(enc_hbm, out_hbm, *scratch):
    scratch = scratch[:_N_TEC_SCRATCH]
    bufs = scratch[:TEC_NBUF]
    in_sems = scratch[TEC_NBUF : 2 * TEC_NBUF]
    out_sems = scratch[2 * TEC_NBUF :]
    c = lax.axis_index("c")
    s = lax.axis_index("s")
    base = c * ROWS_PER_C + s * TEC_ROWS_PER_W

    in_copies = [None] * TEC_NCHUNK
    out_copies = [None] * TEC_NCHUNK

    def _scatter(i):
        b = i % TEC_NBUF
        in_copies[i].wait()
        out_copies[i] = pltpu.async_copy(
            bufs[b].at[pl.ds(0, TEC_CHUNKS[i]), :],
            out_hbm.at[0, pl.ds(base + TEC_OFFS[i], TEC_CHUNKS[i]), :],
            out_sems[b],
        )

    for i in range(TEC_NCHUNK):
        b = i % TEC_NBUF
        if i >= TEC_NBUF:
            out_copies[i - TEC_NBUF].wait()
        in_copies[i] = pltpu.async_copy(
            enc_hbm.at[pl.ds(base + TEC_OFFS[i], TEC_CHUNKS[i]), :],
            bufs[b].at[pl.ds(0, TEC_CHUNKS[i]), :],
            in_sems[b],
        )
        if i >= 1:
            _scatter(i - 1)
    _scatter(TEC_NCHUNK - 1)
    for i in range(max(0, TEC_NCHUNK - TEC_NBUF), TEC_NCHUNK):
        out_copies[i].wait()


def _scs_fn(enc_hbm, out_hbm, *scratch):
    scratch = scratch[_N_TEC_SCRATCH:]
    bufs = scratch[:SCS_NBUF]
    in_sems = scratch[SCS_NBUF : 2 * SCS_NBUF]
    out_sems = scratch[2 * SCS_NBUF :]
    base = lax.axis_index("c") * ROWS_PER_C + TEC_ROWS_PER_C

    in_copies = [None] * SCS_NCHUNK
    out_copies = [None] * SCS_NCHUNK

    def _scatter(i):
        b = i % SCS_NBUF
        in_copies[i].wait()
        out_copies[i] = pltpu.async_copy(
            bufs[b],
            out_hbm.at[0, pl.ds(base + i * SCS_CH, SCS_CH), :],
            out_sems[b],
        )

    for i in range(SCS_NCHUNK):
        b = i % SCS_NBUF
        if i >= SCS_NBUF:
            out_copies[i - SCS_NBUF].wait()
        in_copies[i] = pltpu.async_copy(
            enc_hbm.at[pl.ds(base + i * SCS_CH, SCS_CH), :], bufs[b], in_sems[b]
        )
        if i >= 1:
            _scatter(i - 1)
    _scatter(SCS_NCHUNK - 1)
    for i in range(max(0, SCS_NCHUNK - SCS_NBUF), SCS_NCHUNK):
        out_copies[i].wait()


_slice_copy = plmpmd.mpmd_map(
    [(_smesh, _scs_fn), (_vmesh, _tec_fn)],
    out_types=jax.ShapeDtypeStruct((1, SEQ, D), jnp.float32),
    scratch_types=(
        [(pltpu.VMEM @ _vmesh)((TEC_CHMAX, D), jnp.float32)] * TEC_NBUF
        + [pltpu.SemaphoreType.DMA @ _vmesh] * (2 * TEC_NBUF)
        + [pltpu.VMEM_SHARED((SCS_CH, D), jnp.float32)] * SCS_NBUF
        + [pltpu.SemaphoreType.DMA @ _smesh] * (2 * SCS_NBUF)
    ),
)


def kernel(x, encoding):
    del x
    return _slice_copy(encoding)
